# baseline (speedup 1.0000x reference)
import functools

import jax
import jax.numpy as jnp
from jax import lax
from jax.experimental import pallas as pl
from jax.experimental.pallas import tpu as pltpu
from jax.experimental.pallas import tpu_sc as plsc

_N = 10000
_E = 5000
_NNZ = 320000
_D = 128

_NC = 2
_NS = 16
_NW = _NC * _NS

_NP = 10240
_EP = 5120
_CHUNK = 80
_IDX_ROWS = _NNZ // _CHUNK
_ROWS_W = _IDX_ROWS // _NW
_INIT_ROWS = 1000
_SEG = 64

_mesh = plsc.VectorSubcoreMesh(core_axis_name="c", subcore_axis_name="s")


@functools.partial(
    pl.kernel,
    out_type=(
        jax.ShapeDtypeStruct((_NP,), jnp.float32),
        jax.ShapeDtypeStruct((_EP,), jnp.float32),
    ),
    mesh=_mesh,
    scratch_types=[
        pltpu.VMEM_SHARED((_NP,), jnp.float32),
        pltpu.VMEM((_ROWS_W, _CHUNK), jnp.int32),
        pltpu.VMEM((_CHUNK,), jnp.float32),
        pltpu.SemaphoreType.DMA,
    ],
)
def _degrees(nidx_hbm, hidx_hbm, zeros_hbm, dv_out, de_out,
             acc_sh, idx_v, ones_v, sem):
    c = lax.axis_index("c")
    s = lax.axis_index("s")

    @pl.when(s == 0)
    def _():
        pltpu.sync_copy(zeros_hbm, acc_sh)

    for i in range(_CHUNK // 16):
        ones_v[pl.ds(16 * i, 16)] = jnp.full((16,), 1.0, jnp.float32)
    plsc.subcore_barrier()

    def fire(j):
        pltpu.async_copy(ones_v, acc_sh.at[idx_v.at[j]], sem, add=True)

    def drain(j):
        pltpu.make_async_copy(ones_v, acc_sh.at[idx_v.at[j]], sem).wait()

    grp = 8
    nfull = _ROWS_W // grp
    tail = _ROWS_W - nfull * grp
    for half in range(2):
        w = s * 2 + half

        @pl.when(c == 0)
        def _():
            pltpu.sync_copy(nidx_hbm.at[w], idx_v)

        @pl.when(c == 1)
        def _():
            pltpu.sync_copy(hidx_hbm.at[w], idx_v)

        for q in range(grp):
            fire(q)

        def body(p, carry):
            for q in range(grp):
                fire(grp * p + q)
            for q in range(grp):
                drain(grp * (p - 1) + q)
            return carry

        lax.fori_loop(1, nfull, body, 0)
        for q in range(grp):
            drain(grp * (nfull - 1) + q)
        for q in range(tail):
            fire(grp * nfull + q)
        for q in range(tail):
            drain(grp * nfull + q)

    plsc.subcore_barrier()

    @pl.when((c == 0) & (s == 0))
    def _():
        pltpu.sync_copy(acc_sh, dv_out)

    @pl.when((c == 1) & (s == 0))
    def _():
        pltpu.sync_copy(acc_sh.at[pl.ds(0, _EP)], de_out)


def _make_agg(acc_rows):
    n_init = acc_rows // _INIT_ROWS

    @functools.partial(
        pl.kernel,
        out_type=jax.ShapeDtypeStruct((_NC, acc_rows, _D), jnp.float32),
        mesh=_mesh,
        scratch_types=[
            pltpu.VMEM_SHARED((acc_rows, _D), jnp.float32),
            pltpu.VMEM((_SEG, _CHUNK), jnp.int32),
            pltpu.VMEM((_SEG, _CHUNK), jnp.int32),
            pltpu.VMEM((_CHUNK, _D), jnp.float32),
            pltpu.VMEM((_CHUNK, _D), jnp.float32),
            pltpu.SemaphoreType.DMA,
            pltpu.SemaphoreType.DMA,
            pltpu.SemaphoreType.DMA,
            pltpu.SemaphoreType.DMA,
        ],
    )
    def _agg(tbl_hbm, gidx_hbm, sidx_hbm, zeros_hbm, out_hbm,
             acc_sh, gidx_v, sidx_v, rows0_v, rows1_v,
             gsem0, gsem1, ssem0, ssem1):
        c = lax.axis_index("c")
        s = lax.axis_index("s")
        wid = s * _NC + c

        @pl.when(s < n_init)
        def _():
            pltpu.sync_copy(
                zeros_hbm.at[pl.ds(s * _INIT_ROWS, _INIT_ROWS)],
                acc_sh.at[pl.ds(s * _INIT_ROWS, _INIT_ROWS)])

        plsc.subcore_barrier()

        bufs = (rows0_v, rows1_v)
        gsems = (gsem0, gsem1)
        ssems = (ssem0, ssem1)

        def start_g(j, k):
            pltpu.async_copy(tbl_hbm.at[gidx_v.at[j]], bufs[k], gsems[k])

        def wait_g(j, k):
            pltpu.make_async_copy(tbl_hbm.at[gidx_v.at[j]], bufs[k],
                                  gsems[k]).wait()

        def start_s(j, k):
            pltpu.async_copy(bufs[k], acc_sh.at[sidx_v.at[j]], ssems[k],
                             add=True)

        def wait_s(j, k):
            pltpu.make_async_copy(bufs[k], acc_sh.at[sidx_v.at[j]],
                                  ssems[k]).wait()

        for off, n in ((0, _SEG), (_SEG, _ROWS_W - _SEG)):
            pltpu.sync_copy(gidx_hbm.at[wid, pl.ds(off, n)],
                            gidx_v.at[pl.ds(0, n)])
            pltpu.sync_copy(sidx_hbm.at[wid, pl.ds(off, n)],
                            sidx_v.at[pl.ds(0, n)])
            start_g(0, 0)
            start_g(1, 1)

            def body(p, carry, n=n):
                j = 2 * p
                wait_g(j, 0)
                start_s(j, 0)

                @pl.when(j + 1 < n)
                def _():
                    wait_g(j + 1, 1)
                    start_s(j + 1, 1)

                wait_s(j, 0)

                @pl.when(j + 2 < n)
                def _():
                    start_g(j + 2, 0)

                @pl.when(j + 1 < n)
                def _():
                    wait_s(j + 1, 1)

                @pl.when(j + 3 < n)
                def _():
                    start_g(j + 3, 1)

                return carry

            lax.fori_loop(0, (n + 1) // 2, body, 0)

        plsc.subcore_barrier()

        @pl.when(s < n_init)
        def _():
            pltpu.sync_copy(
                acc_sh.at[pl.ds(s * _INIT_ROWS, _INIT_ROWS)],
                out_hbm.at[c, pl.ds(s * _INIT_ROWS, _INIT_ROWS)])

    return _agg


_agg_edges = _make_agg(_E)
_agg_nodes = _make_agg(_N)


def _proj_body(x_ref, w_ref, b_ref, dv_ref, out_ref):
    h = lax.dot_general(x_ref[...], w_ref[...], (((1,), (1,)), ((), ())),
                        preferred_element_type=jnp.float32)
    dv = dv_ref[...]
    scale = jnp.where(dv > 0, lax.rsqrt(dv), 0.0)
    out_ref[...] = (h + b_ref[...]) * scale


def _edge_body(p_ref, de_ref, out_ref):
    de = de_ref[...]
    inv = jnp.where(de > 0, 1.0 / de, 0.0)
    out_ref[...] = (p_ref[0] + p_ref[1]) * inv


def _node_body(p_ref, dv_ref, out_ref):
    dv = dv_ref[...]
    scale = jnp.where(dv > 0, lax.rsqrt(dv), 0.0)
    out_ref[...] = jnp.maximum((p_ref[0] + p_ref[1]) * scale, 0.0)


def kernel(X, node_idx, hedge_idx, W, b):
    nidx2 = node_idx.astype(jnp.int32).reshape(_NW, _ROWS_W, _CHUNK)
    hidx2 = hedge_idx.astype(jnp.int32).reshape(_NW, _ROWS_W, _CHUNK)
    zeros1 = jnp.zeros((_NP,), jnp.float32)
    zeros2 = jnp.zeros((_N, _D), jnp.float32)

    dvp, dep = _degrees(nidx2, hidx2, zeros1)
    dv = dvp[:_N]
    de = dep[:_E]

    hs = pl.pallas_call(
        _proj_body,
        out_shape=jax.ShapeDtypeStruct((_N, _D), jnp.float32),
    )(X, W, b.reshape(1, _D), dv.reshape(_N, 1))

    yep = _agg_edges(hs, nidx2, hidx2, zeros2)

    ye = pl.pallas_call(
        _edge_body,
        out_shape=jax.ShapeDtypeStruct((_E, _D), jnp.float32),
    )(yep, de.reshape(_E, 1))

    zp = _agg_nodes(ye, hidx2, nidx2, zeros2)

    z = pl.pallas_call(
        _node_body,
        out_shape=jax.ShapeDtypeStruct((_N, _D), jnp.float32),
    )(zp, dv.reshape(_N, 1))
    return z


# speedup vs baseline: 1.2262x; 1.2262x over previous
import functools

import jax
import jax.numpy as jnp
from jax import lax
from jax.experimental import pallas as pl
from jax.experimental.pallas import tpu as pltpu
from jax.experimental.pallas import tpu_sc as plsc

_N = 10000
_E = 5000
_NNZ = 320000
_D = 128

_NC = 2
_NS = 16
_NW = _NC * _NS

_NP = 10240
_EP = 5120
_CHUNK = 80
_IDX_ROWS = _NNZ // _CHUNK
_ROWS_W = _IDX_ROWS // _NW
_INIT_ROWS = 1000
_SEG = 64

_mesh = plsc.VectorSubcoreMesh(core_axis_name="c", subcore_axis_name="s")


@functools.partial(
    pl.kernel,
    out_type=(
        jax.ShapeDtypeStruct((_NP,), jnp.float32),
        jax.ShapeDtypeStruct((_EP,), jnp.float32),
    ),
    mesh=_mesh,
    scratch_types=[
        pltpu.VMEM_SHARED((_NP,), jnp.float32),
        pltpu.VMEM((_ROWS_W, _CHUNK), jnp.int32),
        pltpu.VMEM((_CHUNK,), jnp.float32),
        pltpu.SemaphoreType.DMA,
    ],
)
def _degrees(nidx_hbm, hidx_hbm, zeros_hbm, dv_out, de_out,
             acc_sh, idx_v, ones_v, sem):
    c = lax.axis_index("c")
    s = lax.axis_index("s")

    @pl.when(s == 0)
    def _():
        pltpu.sync_copy(zeros_hbm, acc_sh)

    for i in range(_CHUNK // 16):
        ones_v[pl.ds(16 * i, 16)] = jnp.full((16,), 1.0, jnp.float32)
    plsc.subcore_barrier()

    def fire(j):
        pltpu.async_copy(ones_v, acc_sh.at[idx_v.at[j]], sem, add=True)

    def drain(j):
        pltpu.make_async_copy(ones_v, acc_sh.at[idx_v.at[j]], sem).wait()

    grp = 8
    nfull = _ROWS_W // grp
    tail = _ROWS_W - nfull * grp
    for half in range(2):
        w = s * 2 + half

        @pl.when(c == 0)
        def _():
            pltpu.sync_copy(nidx_hbm.at[w], idx_v)

        @pl.when(c == 1)
        def _():
            pltpu.sync_copy(hidx_hbm.at[w], idx_v)

        for q in range(grp):
            fire(q)

        def body(p, carry):
            for q in range(grp):
                fire(grp * p + q)
            for q in range(grp):
                drain(grp * (p - 1) + q)
            return carry

        lax.fori_loop(1, nfull, body, 0)
        for q in range(grp):
            drain(grp * (nfull - 1) + q)
        for q in range(tail):
            fire(grp * nfull + q)
        for q in range(tail):
            drain(grp * nfull + q)

    plsc.subcore_barrier()

    @pl.when((c == 0) & (s == 0))
    def _():
        pltpu.sync_copy(acc_sh, dv_out)

    @pl.when((c == 1) & (s == 0))
    def _():
        pltpu.sync_copy(acc_sh.at[pl.ds(0, _EP)], de_out)


def _make_agg(acc_rows):
    n_init = acc_rows // _INIT_ROWS

    @functools.partial(
        pl.kernel,
        out_type=jax.ShapeDtypeStruct((_NC, acc_rows, _D), jnp.float32),
        mesh=_mesh,
        scratch_types=[
            pltpu.VMEM_SHARED((acc_rows, _D), jnp.float32),
            pltpu.VMEM((_SEG, _CHUNK), jnp.int32),
            pltpu.VMEM((_SEG, _CHUNK), jnp.int32),
            pltpu.VMEM((_CHUNK, _D), jnp.float32),
            pltpu.VMEM((_CHUNK, _D), jnp.float32),
            pltpu.SemaphoreType.DMA,
            pltpu.SemaphoreType.DMA,
            pltpu.SemaphoreType.DMA,
            pltpu.SemaphoreType.DMA,
        ],
    )
    def _agg(tbl_hbm, gidx_hbm, sidx_hbm, zeros_hbm, out_hbm,
             acc_sh, gidx_v, sidx_v, rows0_v, rows1_v,
             gsem0, gsem1, ssem0, ssem1):
        c = lax.axis_index("c")
        s = lax.axis_index("s")
        wid = s * _NC + c

        @pl.when(s < n_init)
        def _():
            pltpu.sync_copy(
                zeros_hbm.at[pl.ds(s * _INIT_ROWS, _INIT_ROWS)],
                acc_sh.at[pl.ds(s * _INIT_ROWS, _INIT_ROWS)])

        plsc.subcore_barrier()

        bufs = (rows0_v, rows1_v)
        gsems = (gsem0, gsem1)
        ssems = (ssem0, ssem1)

        def start_g(j, k):
            pltpu.async_copy(tbl_hbm.at[gidx_v.at[j]], bufs[k], gsems[k])

        def wait_g(j, k):
            pltpu.make_async_copy(tbl_hbm.at[gidx_v.at[j]], bufs[k],
                                  gsems[k]).wait()

        def scat(j, k):
            pltpu.sync_copy(bufs[k], acc_sh.at[sidx_v.at[j]], add=True)

        for off, n in ((0, _SEG), (_SEG, _ROWS_W - _SEG)):
            pltpu.sync_copy(gidx_hbm.at[wid, pl.ds(off, n)],
                            gidx_v.at[pl.ds(0, n)])
            pltpu.sync_copy(sidx_hbm.at[wid, pl.ds(off, n)],
                            sidx_v.at[pl.ds(0, n)])
            start_g(0, 0)

            def body(p, carry, n=n):
                j = 2 * p

                @pl.when(j + 1 < n)
                def _():
                    start_g(j + 1, 1)

                wait_g(j, 0)
                scat(j, 0)

                @pl.when(j + 2 < n)
                def _():
                    start_g(j + 2, 0)

                @pl.when(j + 1 < n)
                def _():
                    wait_g(j + 1, 1)
                    scat(j + 1, 1)

                return carry

            lax.fori_loop(0, (n + 1) // 2, body, 0)

        plsc.subcore_barrier()

        @pl.when(s < n_init)
        def _():
            pltpu.sync_copy(
                acc_sh.at[pl.ds(s * _INIT_ROWS, _INIT_ROWS)],
                out_hbm.at[c, pl.ds(s * _INIT_ROWS, _INIT_ROWS)])

    return _agg


_agg_edges = _make_agg(_E)
_agg_nodes = _make_agg(_N)


def _proj_body(x_ref, w_ref, b_ref, dv_ref, out_ref):
    h = lax.dot_general(x_ref[...], w_ref[...], (((1,), (1,)), ((), ())),
                        preferred_element_type=jnp.float32)
    dv = dv_ref[...]
    scale = jnp.where(dv > 0, lax.rsqrt(dv), 0.0)
    out_ref[...] = (h + b_ref[...]) * scale


def _edge_body(p_ref, de_ref, out_ref):
    de = de_ref[...]
    inv = jnp.where(de > 0, 1.0 / de, 0.0)
    out_ref[...] = (p_ref[0] + p_ref[1]) * inv


def _node_body(p_ref, dv_ref, out_ref):
    dv = dv_ref[...]
    scale = jnp.where(dv > 0, lax.rsqrt(dv), 0.0)
    out_ref[...] = jnp.maximum((p_ref[0] + p_ref[1]) * scale, 0.0)


def kernel(X, node_idx, hedge_idx, W, b):
    nidx2 = node_idx.astype(jnp.int32).reshape(_NW, _ROWS_W, _CHUNK)
    hidx2 = hedge_idx.astype(jnp.int32).reshape(_NW, _ROWS_W, _CHUNK)
    zeros1 = jnp.zeros((_NP,), jnp.float32)
    zeros2 = jnp.zeros((_N, _D), jnp.float32)

    dvp, dep = _degrees(nidx2, hidx2, zeros1)
    dv = dvp[:_N]
    de = dep[:_E]

    hs = pl.pallas_call(
        _proj_body,
        out_shape=jax.ShapeDtypeStruct((_N, _D), jnp.float32),
    )(X, W, b.reshape(1, _D), dv.reshape(_N, 1))

    yep = _agg_edges(hs, nidx2, hidx2, zeros2)

    ye = pl.pallas_call(
        _edge_body,
        out_shape=jax.ShapeDtypeStruct((_E, _D), jnp.float32),
    )(yep, de.reshape(_E, 1))

    zp = _agg_nodes(ye, hidx2, nidx2, zeros2)

    z = pl.pallas_call(
        _node_body,
        out_shape=jax.ShapeDtypeStruct((_N, _D), jnp.float32),
    )(zp, dv.reshape(_N, 1))
    return z
